# baseline (speedup 1.0000x reference)
import functools

import jax
import jax.numpy as jnp
import numpy as np
from jax.experimental import pallas as pl
from jax.experimental.pallas import tpu as pltpu
from jax.experimental.pallas import tpu_sc as plsc

P = 15
G = 32
SLOPE = 0.1
EPS = 1e-5
K = 32


def _lk(x):
    return jnp.where(x >= 0, x, SLOPE * x)


def _ceil_to(n, m):
    return ((n + m - 1) // m) * m


def _gather_window(e, d):
    cap = 128 if d <= 128 else (64 if d <= 256 else 32)
    for w in range(cap, 0, -8):
        if e % w == 0:
            return w
    return cap


def _sc_gather(table, idx):
    e = idx.shape[0]
    d = table.shape[1]
    window = _gather_window(e, d)
    ep = _ceil_to(e, window)
    if ep != e:
        idx = jnp.concatenate([idx, jnp.zeros((ep - e,), jnp.int32)])
    idx2 = idx.reshape(1, ep)
    mesh = plsc.VectorSubcoreMesh(core_axis_name="core",
                                  subcore_axis_name="subcore")

    @functools.partial(
        pl.kernel,
        out_type=jax.ShapeDtypeStruct((ep, d), table.dtype),
        mesh=mesh,
        compiler_params=pltpu.CompilerParams(use_tc_tiling_on_sc=False),
    )
    def gk(x_hbm, i_hbm, o_hbm):
        def body(i_vmem, o_vmem):
            pltpu.sync_copy(x_hbm.at[i_vmem.at[0]], o_vmem)

        pltpu.emit_pipeline(
            body,
            grid=(ep // window,),
            in_specs=[pl.BlockSpec((1, window), lambda i: (0, i))],
            out_specs=[pl.BlockSpec((window, d), lambda i: (i, 0))],
            core_axis_name=("core", "subcore"),
            dimension_semantics=(pltpu.PARALLEL,),
        )(i_hbm, o_hbm)

    out = gk(table, idx2)
    if ep != e:
        out = out[:e]
    return out


def _m1(c):
    gs = c // G
    return jnp.asarray(np.kron(np.eye(G, dtype=np.float32),
                               np.ones((gs, gs), np.float32)))


def _gb(gn):
    c = gn['g'].shape[0]
    z = jnp.zeros((6, c), jnp.float32)
    return jnp.concatenate([gn['g'].reshape(1, c), gn['be'].reshape(1, c), z], 0)


def _gn_ab(st_ref, gb_ref, m1_ref, count):
    st = jnp.dot(st_ref[...], m1_ref[...], preferred_element_type=jnp.float32,
                 precision=jax.lax.Precision.HIGHEST)
    mean = st[0:1, :] / count
    var = st[1:2, :] / count - mean * mean
    a = gb_ref[0:1, :] * jax.lax.rsqrt(var + EPS)
    b = gb_ref[1:2, :] - mean * a
    return a, b


def _accum_stats(st_ref, o):
    d = o.shape[1]
    s1 = jnp.sum(o, axis=0, keepdims=True)
    s2 = jnp.sum(o * o, axis=0, keepdims=True)
    blk = jnp.concatenate([s1, s2, jnp.zeros((6, d), jnp.float32)], 0)

    @pl.when(pl.program_id(0) == 0)
    def _():
        st_ref[...] = jnp.zeros_like(st_ref)

    st_ref[...] += blk


def _rowmask(blk, nreal, mp):
    if mp == nreal:
        return None
    rows = jax.lax.broadcasted_iota(jnp.int32, (blk, 1), 0) \
        + pl.program_id(0) * blk
    return (rows < nreal).astype(jnp.float32)


def _kcon(kpts):
    kc = jnp.zeros((8, 16), jnp.float32)
    kc = kc.at[0:3, 0:15].set(kpts.T)
    kc = kc.at[3, 0:15].set(jnp.sum(kpts * kpts, axis=1))
    kc = kc.at[3, 15].set(1e12)
    kc = kc.at[4, 0:15].set(1.0)
    return kc


def _edge_wts(pg3, q, kc, sigma, bq, r):
    d3 = pg3[:, :, 0:3] - q[:, None, 0:3]
    d = d3.reshape(r, 3)
    ex = d[:, 0:1] - kc[0:1, :]
    ey = d[:, 1:2] - kc[1:2, :]
    ez = d[:, 2:3] - kc[2:3, :]
    sq = ex * ex + ey * ey + ez * ez
    return jnp.clip(1.0 - jnp.sqrt(sq) / sigma, 0.0, None) * kc[4:5, :]


def _kpconv(z, pg, qp, kpts, sigma, w, mp, nreal, bq, bc, c, d, norm):
    pack = max(1, 128 // c)
    npass = -(-P // pack)
    p_pad = pack * npass
    cpk = c * pack
    wp = jnp.zeros((p_pad, c, d), jnp.float32).at[0:P].set(w).reshape(p_pad * c, d)
    kc_v = _kcon(kpts)
    r = bq * K
    stats, gbv, m1v, count = norm if norm is not None else (None,) * 4
    ej_v = jnp.asarray(
        np.equal(np.arange(cpk)[None, :] // c,
                 np.arange(pack)[:, None]).astype(np.float32))
    tile_v = jnp.asarray(np.tile(np.eye(c, dtype=np.float32), (1, pack)))

    def body(*refs):
        if norm is not None:
            (z_ref, pg_ref, qp_ref, kc_ref, ej_ref, tile_ref, st_in, gb_ref,
             m1_ref, w_ref, o_ref, st_ref) = refs
        else:
            (z_ref, pg_ref, qp_ref, kc_ref, ej_ref, tile_ref, w_ref,
             o_ref, st_ref) = refs
        zv = z_ref[...]
        if norm is not None:
            a, b = _gn_ab(st_in, gb_ref, m1_ref, count)
            zv = _lk(zv * a + b)
        sv = _edge_wts(pg_ref[...].reshape(bq, K, 16), qp_ref[...],
                       kc_ref[...], sigma, bq, r)
        ej = ej_ref[...]
        hs = []
        for t in range(bq // bc):
            rc = bc * K
            z2 = zv[t * rc:(t + 1) * rc, :]
            if pack > 1:
                z2 = jnp.dot(z2, tile_ref[...],
                             preferred_element_type=jnp.float32,
                             precision=jax.lax.Precision.HIGHEST)
            zrep = z2.reshape(bc, K, cpk)
            sc_ = sv[t * rc:(t + 1) * rc, :]
            hts = []
            for j in range(npass):
                srep = jnp.dot(sc_[:, j * pack:(j + 1) * pack], ej,
                               preferred_element_type=jnp.float32,
                               precision=jax.lax.Precision.HIGHEST)
                hts.append(jnp.sum(zrep * srep.reshape(bc, K, cpk), axis=1))
            hs.append(jnp.concatenate(hts, axis=1))
        h = jnp.concatenate(hs, axis=0)
        o = jnp.dot(h, w_ref[...], preferred_element_type=jnp.float32,
                 precision=jax.lax.Precision.HIGHEST)
        m = _rowmask(bq, nreal, mp)
        if m is not None:
            o = o * m
        o_ref[...] = o
        _accum_stats(st_ref, o)

    in_specs = [
        pl.BlockSpec((r, c), lambda i: (i, 0)),
        pl.BlockSpec((r, 16), lambda i: (i, 0)),
        pl.BlockSpec((bq, 16), lambda i: (i, 0)),
        pl.BlockSpec((8, 16), lambda i: (0, 0)),
        pl.BlockSpec((pack, cpk), lambda i: (0, 0)),
        pl.BlockSpec((c, cpk), lambda i: (0, 0)),
    ]
    args = [z, pg, qp, kc_v, ej_v, tile_v]
    if norm is not None:
        in_specs += [
            pl.BlockSpec((8, c), lambda i: (0, 0)),
            pl.BlockSpec((8, c), lambda i: (0, 0)),
            pl.BlockSpec((c, c), lambda i: (0, 0)),
        ]
        args += [stats, gbv, m1v]
    in_specs.append(pl.BlockSpec((p_pad * c, d), lambda i: (0, 0)))
    args.append(wp)
    return pl.pallas_call(
        body,
        grid=(mp // bq,),
        in_specs=in_specs,
        out_specs=[
            pl.BlockSpec((bq, d), lambda i: (i, 0)),
            pl.BlockSpec((8, d), lambda i: (0, 0)),
        ],
        out_shape=[
            jax.ShapeDtypeStruct((mp, d), jnp.float32),
            jax.ShapeDtypeStruct((8, d), jnp.float32),
        ],
    )(*args)


def _kpconv_c0(pg, qp, kpts, sigma, w, m, bq):
    d = w.shape[2]
    w16 = jnp.zeros((16, d), jnp.float32).at[0:P].set(w[:, 0, :])
    kc_v = _kcon(kpts)
    r = bq * K

    def body(pg_ref, qp_ref, kc_ref, w_ref, o_ref, st_ref):
        pg3 = pg_ref[...].reshape(bq, K, 16)
        sv = _edge_wts(pg3, qp_ref[...], kc_ref[...], sigma, bq, r)
        z = pg3[:, :, 3:4]
        s3 = sv.reshape(bq, K, 16)
        h = jnp.sum(jnp.broadcast_to(z, (bq, K, 16)) * s3, axis=1)
        o = jnp.dot(h, w_ref[...], preferred_element_type=jnp.float32,
                 precision=jax.lax.Precision.HIGHEST)
        o_ref[...] = o
        _accum_stats(st_ref, o)

    return pl.pallas_call(
        body,
        grid=(m // bq,),
        in_specs=[
            pl.BlockSpec((r, 16), lambda i: (i, 0)),
            pl.BlockSpec((bq, 16), lambda i: (i, 0)),
            pl.BlockSpec((8, 16), lambda i: (0, 0)),
            pl.BlockSpec((16, d), lambda i: (0, 0)),
        ],
        out_specs=[
            pl.BlockSpec((bq, d), lambda i: (i, 0)),
            pl.BlockSpec((8, d), lambda i: (0, 0)),
        ],
        out_shape=[
            jax.ShapeDtypeStruct((m, d), jnp.float32),
            jax.ShapeDtypeStruct((8, d), jnp.float32),
        ],
    )(pg, qp, kc_v, w16)


def _dense(xs, ws, b, mp, nreal, blk, norm=None, want_stats=True):
    d = ws[0].shape[1]
    b2 = b.reshape(1, d)
    nx = len(xs)
    stats, gbv, m1v, count = norm if norm is not None else (None,) * 4
    cn = xs[0].shape[1]

    def body(*refs):
        i = 0
        x_refs = refs[i:i + nx]; i += nx
        if norm is not None:
            st_in, gb_ref, m1_ref = refs[i:i + 3]; i += 3
        w_refs = refs[i:i + nx]; i += nx
        b_ref = refs[i]; i += 1
        if want_stats:
            o_ref, st_ref = refs[i:i + 2]
        else:
            o_ref = refs[i]
        x0 = x_refs[0][...]
        if norm is not None:
            a, bb = _gn_ab(st_in, gb_ref, m1_ref, count)
            x0 = _lk(x0 * a + bb)
        o = jnp.dot(x0, w_refs[0][...], preferred_element_type=jnp.float32,
                 precision=jax.lax.Precision.HIGHEST)
        for j in range(1, nx):
            o += jnp.dot(x_refs[j][...], w_refs[j][...],
                         preferred_element_type=jnp.float32,
                 precision=jax.lax.Precision.HIGHEST)
        o += b_ref[...]
        m = _rowmask(blk, nreal, mp)
        if m is not None:
            o = o * m
        o_ref[...] = o
        if want_stats:
            _accum_stats(st_ref, o)

    in_specs = [pl.BlockSpec((blk, x.shape[1]), lambda i: (i, 0)) for x in xs]
    args = list(xs)
    if norm is not None:
        in_specs += [
            pl.BlockSpec((8, cn), lambda i: (0, 0)),
            pl.BlockSpec((8, cn), lambda i: (0, 0)),
            pl.BlockSpec((cn, cn), lambda i: (0, 0)),
        ]
        args += [stats, gbv, m1v]
    in_specs += [pl.BlockSpec((w.shape[0], d), lambda i: (0, 0)) for w in ws]
    args += list(ws)
    in_specs.append(pl.BlockSpec((1, d), lambda i: (0, 0)))
    args.append(b2)
    if want_stats:
        out_specs = [
            pl.BlockSpec((blk, d), lambda i: (i, 0)),
            pl.BlockSpec((8, d), lambda i: (0, 0)),
        ]
        out_shape = [
            jax.ShapeDtypeStruct((mp, d), jnp.float32),
            jax.ShapeDtypeStruct((8, d), jnp.float32),
        ]
    else:
        out_specs = pl.BlockSpec((blk, d), lambda i: (i, 0))
        out_shape = jax.ShapeDtypeStruct((mp, d), jnp.float32)
    return pl.pallas_call(
        body,
        grid=(mp // blk,),
        in_specs=in_specs,
        out_specs=out_specs,
        out_shape=out_shape,
    )(*args)


def _finalize(raw, stats, gbv, mp, nreal, blk, count):
    d = raw.shape[1]
    m1v = _m1(d)

    def body(x_ref, st_ref, gb_ref, m1_ref, o_ref):
        a, b = _gn_ab(st_ref, gb_ref, m1_ref, count)
        o = _lk(x_ref[...] * a + b)
        m = _rowmask(blk, nreal, mp)
        if m is not None:
            o = o * m
        o_ref[...] = o

    return pl.pallas_call(
        body,
        grid=(mp // blk,),
        in_specs=[
            pl.BlockSpec((blk, d), lambda i: (i, 0)),
            pl.BlockSpec((8, d), lambda i: (0, 0)),
            pl.BlockSpec((8, d), lambda i: (0, 0)),
            pl.BlockSpec((d, d), lambda i: (0, 0)),
        ],
        out_specs=pl.BlockSpec((blk, d), lambda i: (i, 0)),
        out_shape=jax.ShapeDtypeStruct((mp, d), jnp.float32),
    )(raw, stats, gbv, m1v)


def _combine(fraw, fstats, fgb, sc, scnorm, mp, nreal, blk, count):
    d = fraw.shape[1]
    m1v = _m1(d)

    def body(*refs):
        if scnorm is not None:
            f_ref, fst, fgb_ref, s_ref, sst, sgb_ref, m1_ref, o_ref = refs
        else:
            f_ref, fst, fgb_ref, s_ref, m1_ref, o_ref = refs
        a1, b1 = _gn_ab(fst, fgb_ref, m1_ref, count)
        f = f_ref[...] * a1 + b1
        if scnorm is not None:
            a2, b2 = _gn_ab(sst, sgb_ref, m1_ref, count)
            s = s_ref[...] * a2 + b2
        else:
            s = s_ref[...]
        o = _lk(f + s)
        m = _rowmask(blk, nreal, mp)
        if m is not None:
            o = o * m
        o_ref[...] = o

    in_specs = [
        pl.BlockSpec((blk, d), lambda i: (i, 0)),
        pl.BlockSpec((8, d), lambda i: (0, 0)),
        pl.BlockSpec((8, d), lambda i: (0, 0)),
        pl.BlockSpec((blk, d), lambda i: (i, 0)),
    ]
    args = [fraw, fstats, fgb, sc]
    if scnorm is not None:
        in_specs += [
            pl.BlockSpec((8, d), lambda i: (0, 0)),
            pl.BlockSpec((8, d), lambda i: (0, 0)),
        ]
        args += list(scnorm)
    in_specs.append(pl.BlockSpec((d, d), lambda i: (0, 0)))
    args.append(m1v)
    return pl.pallas_call(
        body,
        grid=(mp // blk,),
        in_specs=in_specs,
        out_specs=pl.BlockSpec((blk, d), lambda i: (i, 0)),
        out_shape=jax.ShapeDtypeStruct((mp, d), jnp.float32),
    )(*args)


def _maxpool(zg, mp, nreal, bq, c):
    r = bq * K

    def body(z_ref, o_ref):
        z3 = z_ref[...].reshape(bq, K, c)
        o = jnp.max(z3, axis=1)
        m = _rowmask(bq, nreal, mp)
        if m is not None:
            o = o * m
        o_ref[...] = o

    return pl.pallas_call(
        body,
        grid=(mp // bq,),
        in_specs=[pl.BlockSpec((r, c), lambda i: (i, 0))],
        out_specs=pl.BlockSpec((bq, c), lambda i: (i, 0)),
        out_shape=jax.ShapeDtypeStruct((mp, c), jnp.float32),
    )(zg)


_LV = {
    0: dict(n=10000, mp=10000, blk=1000, bq=200, bc=200),
    1: dict(n=2500, mp=2560, blk=512, bq=128, bc=128),
    2: dict(n=625, mp=640, blk=640, bq=128, bc=128),
}


def _res_block(x, pg, qp, idx, p, sigma, lq, ls, strided):
    q = _LV[lq]
    s = _LV[ls]
    cin = x.shape[1]
    mid = p['u1']['lin']['w'].shape[0]
    cout = p['u2']['lin']['w'].shape[0]
    u1raw, u1st = _dense([x], [p['u1']['lin']['w'].T], p['u1']['lin']['b'],
                         s['mp'], s['n'], s['blk'])
    z = _sc_gather(u1raw, idx)
    kpraw, kpst = _kpconv(z, pg, qp, p['kp']['kpts'], sigma, p['kp']['w'],
                          q['mp'], q['n'], q['bq'], q['bc'], mid, mid,
                          (u1st, _gb(p['u1']['gn']), _m1(mid),
                           float(s['n'] * (mid // G))))
    u2raw, u2st = _dense([kpraw], [p['u2']['lin']['w'].T], p['u2']['lin']['b'],
                         q['mp'], q['n'], q['blk'],
                         norm=(kpst, _gb(p['gnc']), _m1(mid),
                               float(q['n'] * (mid // G))))
    count = float(q['n'] * (cout // G))
    if strided:
        zm = _sc_gather(x, idx)
        scf = _maxpool(zm, q['mp'], q['n'], q['bq'], cin)
        return _combine(u2raw, u2st, _gb(p['u2']['gn']), scf, None,
                        q['mp'], q['n'], q['blk'], count)
    scraw, scst = _dense([x], [p['usc']['lin']['w'].T], p['usc']['lin']['b'],
                         q['mp'], q['n'], q['blk'])
    return _combine(u2raw, u2st, _gb(p['u2']['gn']), scraw,
                    (scst, _gb(p['usc']['gn'])), q['mp'], q['n'], q['blk'],
                    count)


def _pad_rows(a, mp):
    n = a.shape[0]
    if mp == n:
        return a
    return jnp.concatenate(
        [a, jnp.zeros((mp - n,) + a.shape[1:], a.dtype)], axis=0)


def kernel(feats, points_0, points_1, points_2, neighbors_0, neighbors_1,
           neighbors_2, subsampling_0, subsampling_1, upsampling_0,
           upsampling_1, params):
    n0 = points_0.shape[0]

    t0 = jnp.concatenate([points_0, feats,
                          jnp.zeros((n0, 12), jnp.float32)], axis=1)
    t1 = _pad_rows(jnp.concatenate(
        [points_1, jnp.zeros((_LV[1]['n'], 13), jnp.float32)], axis=1),
        _LV[1]['mp'])
    t2 = _pad_rows(jnp.concatenate(
        [points_2, jnp.zeros((_LV[2]['n'], 13), jnp.float32)], axis=1),
        _LV[2]['mp'])

    i_n0 = neighbors_0.reshape(-1)
    i_s0 = _pad_rows(subsampling_0.reshape(-1), _LV[1]['mp'] * K)
    i_n1 = _pad_rows(neighbors_1.reshape(-1), _LV[1]['mp'] * K)
    i_s1 = _pad_rows(subsampling_1.reshape(-1), _LV[2]['mp'] * K)
    i_n2 = _pad_rows(neighbors_2.reshape(-1), _LV[2]['mp'] * K)
    i_u1 = _pad_rows(upsampling_1[:, 0], _LV[1]['mp'])
    i_u0 = upsampling_0[:, 0]

    pg0 = _sc_gather(t0, i_n0)
    pgs0 = _sc_gather(t0, i_s0)
    pg1 = _sc_gather(t1, i_n1)
    pgs1 = _sc_gather(t1, i_s1)
    pg2 = _sc_gather(t2, i_n2)

    pr = params
    c0raw, c0st = _kpconv_c0(pg0, t0, pr['c0']['kp']['kpts'], 0.1,
                             pr['c0']['kp']['w'], n0, 200)
    x = _finalize(c0raw, c0st, _gb(pr['c0']['gn']), n0, n0, 1000,
                  float(n0 * (64 // G)))

    x = _res_block(x, pg0, t0, i_n0, pr['r01'], 0.1, 0, 0, False)
    e0 = x
    x = _res_block(x, pgs0, t1, i_s0, pr['r10'], 0.1, 1, 0, True)
    x = _res_block(x, pg1, t1, i_n1, pr['r11'], 0.2, 1, 1, False)
    e1 = x
    x = _res_block(x, pgs1, t2, i_s1, pr['r20'], 0.2, 2, 1, True)
    x = _res_block(x, pg2, t2, i_n2, pr['r21'], 0.4, 2, 2, False)
    e2 = x

    u1g = _sc_gather(e2, i_u1)
    d0w = pr['d0']['lin']['w']
    d0raw, d0st = _dense([u1g, e1], [d0w[:, :512].T, d0w[:, 512:].T],
                         pr['d0']['lin']['b'], _LV[1]['mp'], _LV[1]['n'],
                         _LV[1]['blk'])
    d_mid = _finalize(d0raw, d0st, _gb(pr['d0']['gn']), _LV[1]['mp'],
                      _LV[1]['n'], _LV[1]['blk'],
                      float(_LV[1]['n'] * (256 // G)))

    u0g = _sc_gather(d_mid, i_u0)
    d1w = pr['d1']['lin']['w']
    d_fine = _dense([u0g, e0], [d1w[:, :256].T, d1w[:, 256:].T],
                    pr['d1']['lin']['b'], n0, n0, 1000, want_stats=False)

    return (d_fine, d_mid[:_LV[1]['n']], e2[:_LV[2]['n']])


# speedup vs baseline: 1.0000x; 1.0000x over previous
import functools

import jax
import jax.numpy as jnp
import numpy as np
from jax.experimental import pallas as pl
from jax.experimental.pallas import tpu as pltpu
from jax.experimental.pallas import tpu_sc as plsc

P = 15
G = 32
SLOPE = 0.1
EPS = 1e-5
K = 32


def _lk(x):
    return jnp.where(x >= 0, x, SLOPE * x)


def _ceil_to(n, m):
    return ((n + m - 1) // m) * m


def _gather_window(e, d):
    cap = 128 if d <= 128 else (64 if d <= 256 else 32)
    for w in range(cap, 0, -8):
        if e % w == 0:
            return w
    return cap


def _sc_gather(table, idx):
    e = idx.shape[0]
    d = table.shape[1]
    mesh = plsc.VectorSubcoreMesh(core_axis_name="core",
                                  subcore_axis_name="subcore")
    cp = pltpu.CompilerParams(use_tc_tiling_on_sc=False)

    if d <= 32:
        ep = _ceil_to(e, 1024)
        if ep != e:
            idx = jnp.concatenate([idx, jnp.zeros((ep - e,), jnp.int32)])
        idx2 = idx.reshape(ep // 128, 128)

        @functools.partial(
            pl.kernel,
            out_type=jax.ShapeDtypeStruct((ep, d), table.dtype),
            mesh=mesh,
            compiler_params=cp,
        )
        def gk8(x_hbm, i_hbm, o_hbm):
            def body(i_vmem, o_vmem):
                for t in range(8):
                    pltpu.sync_copy(x_hbm.at[i_vmem.at[t]],
                                    o_vmem.at[pl.ds(t * 128, 128)])

            pltpu.emit_pipeline(
                body,
                grid=(ep // 1024,),
                in_specs=[pl.BlockSpec((8, 128), lambda i: (i, 0))],
                out_specs=[pl.BlockSpec((1024, d), lambda i: (i, 0))],
                core_axis_name=("core", "subcore"),
                dimension_semantics=(pltpu.PARALLEL,),
            )(i_hbm, o_hbm)

        return gk8(table, idx2)

    window = _gather_window(e, d)
    ep = _ceil_to(e, window)
    if ep != e:
        idx = jnp.concatenate([idx, jnp.zeros((ep - e,), jnp.int32)])
    idx2 = idx.reshape(1, ep)

    @functools.partial(
        pl.kernel,
        out_type=jax.ShapeDtypeStruct((ep, d), table.dtype),
        mesh=mesh,
        compiler_params=cp,
    )
    def gk(x_hbm, i_hbm, o_hbm):
        def body(i_vmem, o_vmem):
            pltpu.sync_copy(x_hbm.at[i_vmem.at[0]], o_vmem)

        pltpu.emit_pipeline(
            body,
            grid=(ep // window,),
            in_specs=[pl.BlockSpec((1, window), lambda i: (0, i))],
            out_specs=[pl.BlockSpec((window, d), lambda i: (i, 0))],
            core_axis_name=("core", "subcore"),
            dimension_semantics=(pltpu.PARALLEL,),
        )(i_hbm, o_hbm)

    out = gk(table, idx2)
    if ep != e:
        out = out[:e]
    return out


def _m1(c):
    gs = c // G
    return jnp.asarray(np.kron(np.eye(G, dtype=np.float32),
                               np.ones((gs, gs), np.float32)))


def _gb(gn):
    c = gn['g'].shape[0]
    z = jnp.zeros((6, c), jnp.float32)
    return jnp.concatenate([gn['g'].reshape(1, c), gn['be'].reshape(1, c), z], 0)


def _gn_ab(st_ref, gb_ref, m1_ref, count):
    st = jnp.dot(st_ref[...], m1_ref[...], preferred_element_type=jnp.float32,
                 precision=jax.lax.Precision.HIGHEST)
    mean = st[0:1, :] / count
    var = st[1:2, :] / count - mean * mean
    a = gb_ref[0:1, :] * jax.lax.rsqrt(var + EPS)
    b = gb_ref[1:2, :] - mean * a
    return a, b


def _accum_stats(st_ref, o):
    d = o.shape[1]
    s1 = jnp.sum(o, axis=0, keepdims=True)
    s2 = jnp.sum(o * o, axis=0, keepdims=True)
    blk = jnp.concatenate([s1, s2, jnp.zeros((6, d), jnp.float32)], 0)

    @pl.when(pl.program_id(0) == 0)
    def _():
        st_ref[...] = jnp.zeros_like(st_ref)

    st_ref[...] += blk


def _rowmask(blk, nreal, mp):
    if mp == nreal:
        return None
    rows = jax.lax.broadcasted_iota(jnp.int32, (blk, 1), 0) \
        + pl.program_id(0) * blk
    return (rows < nreal).astype(jnp.float32)


def _kcon(kpts):
    kc = jnp.zeros((8, 16), jnp.float32)
    kc = kc.at[0:3, 0:15].set(kpts.T)
    kc = kc.at[3, 0:15].set(jnp.sum(kpts * kpts, axis=1))
    kc = kc.at[3, 15].set(1e12)
    kc = kc.at[4, 0:15].set(1.0)
    return kc


def _edge_wts(pg3, q, kc, sigma, bq, r):
    d3 = pg3[:, :, 0:3] - q[:, None, 0:3]
    d = d3.reshape(r, 3)
    ex = d[:, 0:1] - kc[0:1, :]
    ey = d[:, 1:2] - kc[1:2, :]
    ez = d[:, 2:3] - kc[2:3, :]
    sq = ex * ex + ey * ey + ez * ez
    return jnp.clip(1.0 - jnp.sqrt(sq) / sigma, 0.0, None) * kc[4:5, :]


def _kpconv(z, pg, qp, kpts, sigma, w, mp, nreal, bq, bc, c, d, norm):
    pack = max(1, 128 // c)
    npass = -(-P // pack)
    p_pad = pack * npass
    cpk = c * pack
    wp = jnp.zeros((p_pad, c, d), jnp.float32).at[0:P].set(w).reshape(p_pad * c, d)
    kc_v = _kcon(kpts)
    r = bq * K
    stats, gbv, m1v, count = norm if norm is not None else (None,) * 4
    ej_v = jnp.asarray(
        np.equal(np.arange(cpk)[None, :] // c,
                 np.arange(pack)[:, None]).astype(np.float32))
    tile_v = jnp.asarray(np.tile(np.eye(c, dtype=np.float32), (1, pack)))

    def body(*refs):
        if norm is not None:
            (z_ref, pg_ref, qp_ref, kc_ref, ej_ref, tile_ref, st_in, gb_ref,
             m1_ref, w_ref, o_ref, st_ref) = refs
        else:
            (z_ref, pg_ref, qp_ref, kc_ref, ej_ref, tile_ref, w_ref,
             o_ref, st_ref) = refs
        zv = z_ref[...]
        if norm is not None:
            a, b = _gn_ab(st_in, gb_ref, m1_ref, count)
            zv = _lk(zv * a + b)
        sv = _edge_wts(pg_ref[...].reshape(bq, K, 16), qp_ref[...],
                       kc_ref[...], sigma, bq, r)
        ej = ej_ref[...]
        hs = []
        for t in range(bq // bc):
            rc = bc * K
            z2 = zv[t * rc:(t + 1) * rc, :]
            if pack > 1:
                z2 = jnp.dot(z2, tile_ref[...],
                             preferred_element_type=jnp.float32,
                             precision=jax.lax.Precision.HIGHEST)
            zrep = z2.reshape(bc, K, cpk)
            sc_ = sv[t * rc:(t + 1) * rc, :]
            hts = []
            for j in range(npass):
                srep = jnp.dot(sc_[:, j * pack:(j + 1) * pack], ej,
                               preferred_element_type=jnp.float32,
                               precision=jax.lax.Precision.HIGHEST)
                hts.append(jnp.sum(zrep * srep.reshape(bc, K, cpk), axis=1))
            hs.append(jnp.concatenate(hts, axis=1))
        h = jnp.concatenate(hs, axis=0)
        o = jnp.dot(h, w_ref[...], preferred_element_type=jnp.float32,
                 precision=jax.lax.Precision.HIGHEST)
        m = _rowmask(bq, nreal, mp)
        if m is not None:
            o = o * m
        o_ref[...] = o
        _accum_stats(st_ref, o)

    in_specs = [
        pl.BlockSpec((r, c), lambda i: (i, 0)),
        pl.BlockSpec((r, 16), lambda i: (i, 0)),
        pl.BlockSpec((bq, 16), lambda i: (i, 0)),
        pl.BlockSpec((8, 16), lambda i: (0, 0)),
        pl.BlockSpec((pack, cpk), lambda i: (0, 0)),
        pl.BlockSpec((c, cpk), lambda i: (0, 0)),
    ]
    args = [z, pg, qp, kc_v, ej_v, tile_v]
    if norm is not None:
        in_specs += [
            pl.BlockSpec((8, c), lambda i: (0, 0)),
            pl.BlockSpec((8, c), lambda i: (0, 0)),
            pl.BlockSpec((c, c), lambda i: (0, 0)),
        ]
        args += [stats, gbv, m1v]
    in_specs.append(pl.BlockSpec((p_pad * c, d), lambda i: (0, 0)))
    args.append(wp)
    return pl.pallas_call(
        body,
        grid=(mp // bq,),
        in_specs=in_specs,
        out_specs=[
            pl.BlockSpec((bq, d), lambda i: (i, 0)),
            pl.BlockSpec((8, d), lambda i: (0, 0)),
        ],
        out_shape=[
            jax.ShapeDtypeStruct((mp, d), jnp.float32),
            jax.ShapeDtypeStruct((8, d), jnp.float32),
        ],
    )(*args)


def _kpconv_c0(pg, qp, kpts, sigma, w, m, bq):
    d = w.shape[2]
    w16 = jnp.zeros((16, d), jnp.float32).at[0:P].set(w[:, 0, :])
    kc_v = _kcon(kpts)
    r = bq * K

    def body(pg_ref, qp_ref, kc_ref, w_ref, o_ref, st_ref):
        pg3 = pg_ref[...].reshape(bq, K, 16)
        sv = _edge_wts(pg3, qp_ref[...], kc_ref[...], sigma, bq, r)
        z = pg3[:, :, 3:4]
        s3 = sv.reshape(bq, K, 16)
        h = jnp.sum(jnp.broadcast_to(z, (bq, K, 16)) * s3, axis=1)
        o = jnp.dot(h, w_ref[...], preferred_element_type=jnp.float32,
                 precision=jax.lax.Precision.HIGHEST)
        o_ref[...] = o
        _accum_stats(st_ref, o)

    return pl.pallas_call(
        body,
        grid=(m // bq,),
        in_specs=[
            pl.BlockSpec((r, 16), lambda i: (i, 0)),
            pl.BlockSpec((bq, 16), lambda i: (i, 0)),
            pl.BlockSpec((8, 16), lambda i: (0, 0)),
            pl.BlockSpec((16, d), lambda i: (0, 0)),
        ],
        out_specs=[
            pl.BlockSpec((bq, d), lambda i: (i, 0)),
            pl.BlockSpec((8, d), lambda i: (0, 0)),
        ],
        out_shape=[
            jax.ShapeDtypeStruct((m, d), jnp.float32),
            jax.ShapeDtypeStruct((8, d), jnp.float32),
        ],
    )(pg, qp, kc_v, w16)


def _dense(xs, ws, b, mp, nreal, blk, norm=None, want_stats=True):
    d = ws[0].shape[1]
    b2 = b.reshape(1, d)
    nx = len(xs)
    stats, gbv, m1v, count = norm if norm is not None else (None,) * 4
    cn = xs[0].shape[1]

    def body(*refs):
        i = 0
        x_refs = refs[i:i + nx]; i += nx
        if norm is not None:
            st_in, gb_ref, m1_ref = refs[i:i + 3]; i += 3
        w_refs = refs[i:i + nx]; i += nx
        b_ref = refs[i]; i += 1
        if want_stats:
            o_ref, st_ref = refs[i:i + 2]
        else:
            o_ref = refs[i]
        x0 = x_refs[0][...]
        if norm is not None:
            a, bb = _gn_ab(st_in, gb_ref, m1_ref, count)
            x0 = _lk(x0 * a + bb)
        o = jnp.dot(x0, w_refs[0][...], preferred_element_type=jnp.float32,
                 precision=jax.lax.Precision.HIGHEST)
        for j in range(1, nx):
            o += jnp.dot(x_refs[j][...], w_refs[j][...],
                         preferred_element_type=jnp.float32,
                 precision=jax.lax.Precision.HIGHEST)
        o += b_ref[...]
        m = _rowmask(blk, nreal, mp)
        if m is not None:
            o = o * m
        o_ref[...] = o
        if want_stats:
            _accum_stats(st_ref, o)

    in_specs = [pl.BlockSpec((blk, x.shape[1]), lambda i: (i, 0)) for x in xs]
    args = list(xs)
    if norm is not None:
        in_specs += [
            pl.BlockSpec((8, cn), lambda i: (0, 0)),
            pl.BlockSpec((8, cn), lambda i: (0, 0)),
            pl.BlockSpec((cn, cn), lambda i: (0, 0)),
        ]
        args += [stats, gbv, m1v]
    in_specs += [pl.BlockSpec((w.shape[0], d), lambda i: (0, 0)) for w in ws]
    args += list(ws)
    in_specs.append(pl.BlockSpec((1, d), lambda i: (0, 0)))
    args.append(b2)
    if want_stats:
        out_specs = [
            pl.BlockSpec((blk, d), lambda i: (i, 0)),
            pl.BlockSpec((8, d), lambda i: (0, 0)),
        ]
        out_shape = [
            jax.ShapeDtypeStruct((mp, d), jnp.float32),
            jax.ShapeDtypeStruct((8, d), jnp.float32),
        ]
    else:
        out_specs = pl.BlockSpec((blk, d), lambda i: (i, 0))
        out_shape = jax.ShapeDtypeStruct((mp, d), jnp.float32)
    return pl.pallas_call(
        body,
        grid=(mp // blk,),
        in_specs=in_specs,
        out_specs=out_specs,
        out_shape=out_shape,
    )(*args)


def _finalize(raw, stats, gbv, mp, nreal, blk, count):
    d = raw.shape[1]
    m1v = _m1(d)

    def body(x_ref, st_ref, gb_ref, m1_ref, o_ref):
        a, b = _gn_ab(st_ref, gb_ref, m1_ref, count)
        o = _lk(x_ref[...] * a + b)
        m = _rowmask(blk, nreal, mp)
        if m is not None:
            o = o * m
        o_ref[...] = o

    return pl.pallas_call(
        body,
        grid=(mp // blk,),
        in_specs=[
            pl.BlockSpec((blk, d), lambda i: (i, 0)),
            pl.BlockSpec((8, d), lambda i: (0, 0)),
            pl.BlockSpec((8, d), lambda i: (0, 0)),
            pl.BlockSpec((d, d), lambda i: (0, 0)),
        ],
        out_specs=pl.BlockSpec((blk, d), lambda i: (i, 0)),
        out_shape=jax.ShapeDtypeStruct((mp, d), jnp.float32),
    )(raw, stats, gbv, m1v)


def _combine(fraw, fstats, fgb, sc, scnorm, mp, nreal, blk, count):
    d = fraw.shape[1]
    m1v = _m1(d)

    def body(*refs):
        if scnorm is not None:
            f_ref, fst, fgb_ref, s_ref, sst, sgb_ref, m1_ref, o_ref = refs
        else:
            f_ref, fst, fgb_ref, s_ref, m1_ref, o_ref = refs
        a1, b1 = _gn_ab(fst, fgb_ref, m1_ref, count)
        f = f_ref[...] * a1 + b1
        if scnorm is not None:
            a2, b2 = _gn_ab(sst, sgb_ref, m1_ref, count)
            s = s_ref[...] * a2 + b2
        else:
            s = s_ref[...]
        o = _lk(f + s)
        m = _rowmask(blk, nreal, mp)
        if m is not None:
            o = o * m
        o_ref[...] = o

    in_specs = [
        pl.BlockSpec((blk, d), lambda i: (i, 0)),
        pl.BlockSpec((8, d), lambda i: (0, 0)),
        pl.BlockSpec((8, d), lambda i: (0, 0)),
        pl.BlockSpec((blk, d), lambda i: (i, 0)),
    ]
    args = [fraw, fstats, fgb, sc]
    if scnorm is not None:
        in_specs += [
            pl.BlockSpec((8, d), lambda i: (0, 0)),
            pl.BlockSpec((8, d), lambda i: (0, 0)),
        ]
        args += list(scnorm)
    in_specs.append(pl.BlockSpec((d, d), lambda i: (0, 0)))
    args.append(m1v)
    return pl.pallas_call(
        body,
        grid=(mp // blk,),
        in_specs=in_specs,
        out_specs=pl.BlockSpec((blk, d), lambda i: (i, 0)),
        out_shape=jax.ShapeDtypeStruct((mp, d), jnp.float32),
    )(*args)


def _maxpool(zg, mp, nreal, bq, c):
    r = bq * K

    def body(z_ref, o_ref):
        z3 = z_ref[...].reshape(bq, K, c)
        o = jnp.max(z3, axis=1)
        m = _rowmask(bq, nreal, mp)
        if m is not None:
            o = o * m
        o_ref[...] = o

    return pl.pallas_call(
        body,
        grid=(mp // bq,),
        in_specs=[pl.BlockSpec((r, c), lambda i: (i, 0))],
        out_specs=pl.BlockSpec((bq, c), lambda i: (i, 0)),
        out_shape=jax.ShapeDtypeStruct((mp, c), jnp.float32),
    )(zg)


_LV = {
    0: dict(n=10000, mp=10000, blk=1000, bq=200, bc=200),
    1: dict(n=2500, mp=2560, blk=512, bq=128, bc=128),
    2: dict(n=625, mp=640, blk=640, bq=128, bc=128),
}


def _res_block(x, pg, qp, idx, p, sigma, lq, ls, strided):
    q = _LV[lq]
    s = _LV[ls]
    cin = x.shape[1]
    mid = p['u1']['lin']['w'].shape[0]
    cout = p['u2']['lin']['w'].shape[0]
    u1raw, u1st = _dense([x], [p['u1']['lin']['w'].T], p['u1']['lin']['b'],
                         s['mp'], s['n'], s['blk'])
    z = _sc_gather(u1raw, idx)
    kpraw, kpst = _kpconv(z, pg, qp, p['kp']['kpts'], sigma, p['kp']['w'],
                          q['mp'], q['n'], q['bq'], q['bc'], mid, mid,
                          (u1st, _gb(p['u1']['gn']), _m1(mid),
                           float(s['n'] * (mid // G))))
    u2raw, u2st = _dense([kpraw], [p['u2']['lin']['w'].T], p['u2']['lin']['b'],
                         q['mp'], q['n'], q['blk'],
                         norm=(kpst, _gb(p['gnc']), _m1(mid),
                               float(q['n'] * (mid // G))))
    count = float(q['n'] * (cout // G))
    if strided:
        zm = _sc_gather(x, idx)
        scf = _maxpool(zm, q['mp'], q['n'], q['bq'], cin)
        return _combine(u2raw, u2st, _gb(p['u2']['gn']), scf, None,
                        q['mp'], q['n'], q['blk'], count)
    scraw, scst = _dense([x], [p['usc']['lin']['w'].T], p['usc']['lin']['b'],
                         q['mp'], q['n'], q['blk'])
    return _combine(u2raw, u2st, _gb(p['u2']['gn']), scraw,
                    (scst, _gb(p['usc']['gn'])), q['mp'], q['n'], q['blk'],
                    count)


def _pad_rows(a, mp):
    n = a.shape[0]
    if mp == n:
        return a
    return jnp.concatenate(
        [a, jnp.zeros((mp - n,) + a.shape[1:], a.dtype)], axis=0)


def kernel(feats, points_0, points_1, points_2, neighbors_0, neighbors_1,
           neighbors_2, subsampling_0, subsampling_1, upsampling_0,
           upsampling_1, params):
    n0 = points_0.shape[0]

    t0 = jnp.concatenate([points_0, feats,
                          jnp.zeros((n0, 12), jnp.float32)], axis=1)
    t1 = _pad_rows(jnp.concatenate(
        [points_1, jnp.zeros((_LV[1]['n'], 13), jnp.float32)], axis=1),
        _LV[1]['mp'])
    t2 = _pad_rows(jnp.concatenate(
        [points_2, jnp.zeros((_LV[2]['n'], 13), jnp.float32)], axis=1),
        _LV[2]['mp'])

    i_n0 = neighbors_0.reshape(-1)
    i_s0 = _pad_rows(subsampling_0.reshape(-1), _LV[1]['mp'] * K)
    i_n1 = _pad_rows(neighbors_1.reshape(-1), _LV[1]['mp'] * K)
    i_s1 = _pad_rows(subsampling_1.reshape(-1), _LV[2]['mp'] * K)
    i_n2 = _pad_rows(neighbors_2.reshape(-1), _LV[2]['mp'] * K)
    i_u1 = _pad_rows(upsampling_1[:, 0], _LV[1]['mp'])
    i_u0 = upsampling_0[:, 0]

    pg0 = _sc_gather(t0, i_n0)
    pgs0 = _sc_gather(t0, i_s0)
    pg1 = _sc_gather(t1, i_n1)
    pgs1 = _sc_gather(t1, i_s1)
    pg2 = _sc_gather(t2, i_n2)

    pr = params
    c0raw, c0st = _kpconv_c0(pg0, t0, pr['c0']['kp']['kpts'], 0.1,
                             pr['c0']['kp']['w'], n0, 200)
    x = _finalize(c0raw, c0st, _gb(pr['c0']['gn']), n0, n0, 1000,
                  float(n0 * (64 // G)))

    x = _res_block(x, pg0, t0, i_n0, pr['r01'], 0.1, 0, 0, False)
    e0 = x
    x = _res_block(x, pgs0, t1, i_s0, pr['r10'], 0.1, 1, 0, True)
    x = _res_block(x, pg1, t1, i_n1, pr['r11'], 0.2, 1, 1, False)
    e1 = x
    x = _res_block(x, pgs1, t2, i_s1, pr['r20'], 0.2, 2, 1, True)
    x = _res_block(x, pg2, t2, i_n2, pr['r21'], 0.4, 2, 2, False)
    e2 = x

    u1g = _sc_gather(e2, i_u1)
    d0w = pr['d0']['lin']['w']
    d0raw, d0st = _dense([u1g, e1], [d0w[:, :512].T, d0w[:, 512:].T],
                         pr['d0']['lin']['b'], _LV[1]['mp'], _LV[1]['n'],
                         _LV[1]['blk'])
    d_mid = _finalize(d0raw, d0st, _gb(pr['d0']['gn']), _LV[1]['mp'],
                      _LV[1]['n'], _LV[1]['blk'],
                      float(_LV[1]['n'] * (256 // G)))

    u0g = _sc_gather(d_mid, i_u0)
    d1w = pr['d1']['lin']['w']
    d_fine = _dense([u0g, e0], [d1w[:, :256].T, d1w[:, 256:].T],
                    pr['d1']['lin']['b'], n0, n0, 1000, want_stats=False)

    return (d_fine, d_mid[:_LV[1]['n']], e2[:_LV[2]['n']])


# speedup vs baseline: 1.0028x; 1.0027x over previous
import functools

import jax
import jax.numpy as jnp
import numpy as np
from jax.experimental import pallas as pl
from jax.experimental.pallas import tpu as pltpu
from jax.experimental.pallas import tpu_sc as plsc

P = 15
G = 32
SLOPE = 0.1
EPS = 1e-5
K = 32


def _lk(x):
    return jnp.where(x >= 0, x, SLOPE * x)


def _ceil_to(n, m):
    return ((n + m - 1) // m) * m


def _gather_window(e, d):
    cap = 128 if d <= 128 else (64 if d <= 256 else 32)
    for w in range(cap, 0, -8):
        if e % w == 0:
            return w
    return cap


def _sc_gather(table, idx):
    e = idx.shape[0]
    d = table.shape[1]
    mesh = plsc.VectorSubcoreMesh(core_axis_name="core",
                                  subcore_axis_name="subcore")
    cp = pltpu.CompilerParams(use_tc_tiling_on_sc=False)

    if d <= 32:
        ep = _ceil_to(e, 1024)
        if ep != e:
            idx = jnp.concatenate([idx, jnp.zeros((ep - e,), jnp.int32)])
        idx2 = idx.reshape(ep // 128, 128)

        @functools.partial(
            pl.kernel,
            out_type=jax.ShapeDtypeStruct((ep, d), table.dtype),
            mesh=mesh,
            compiler_params=cp,
        )
        def gk8(x_hbm, i_hbm, o_hbm):
            def body(i_vmem, o_vmem):
                for t in range(8):
                    pltpu.sync_copy(x_hbm.at[i_vmem.at[t]],
                                    o_vmem.at[pl.ds(t * 128, 128)])

            pltpu.emit_pipeline(
                body,
                grid=(ep // 1024,),
                in_specs=[pl.BlockSpec((8, 128), lambda i: (i, 0))],
                out_specs=[pl.BlockSpec((1024, d), lambda i: (i, 0))],
                core_axis_name=("core", "subcore"),
                dimension_semantics=(pltpu.PARALLEL,),
            )(i_hbm, o_hbm)

        return gk8(table, idx2)

    window = _gather_window(e, d)
    ep = _ceil_to(e, window)
    if ep != e:
        idx = jnp.concatenate([idx, jnp.zeros((ep - e,), jnp.int32)])
    idx2 = idx.reshape(1, ep)

    @functools.partial(
        pl.kernel,
        out_type=jax.ShapeDtypeStruct((ep, d), table.dtype),
        mesh=mesh,
        compiler_params=cp,
    )
    def gk(x_hbm, i_hbm, o_hbm):
        def body(i_vmem, o_vmem):
            pltpu.sync_copy(x_hbm.at[i_vmem.at[0]], o_vmem)

        pltpu.emit_pipeline(
            body,
            grid=(ep // window,),
            in_specs=[pl.BlockSpec((1, window), lambda i: (0, i))],
            out_specs=[pl.BlockSpec((window, d), lambda i: (i, 0))],
            core_axis_name=("core", "subcore"),
            dimension_semantics=(pltpu.PARALLEL,),
        )(i_hbm, o_hbm)

    out = gk(table, idx2)
    if ep != e:
        out = out[:e]
    return out


def _m1(c):
    gs = c // G
    return jnp.asarray(np.kron(np.eye(G, dtype=np.float32),
                               np.ones((gs, gs), np.float32)))


def _gb(gn):
    c = gn['g'].shape[0]
    z = jnp.zeros((6, c), jnp.float32)
    return jnp.concatenate([gn['g'].reshape(1, c), gn['be'].reshape(1, c), z], 0)


def _gn_ab(st_ref, gb_ref, m1_ref, count):
    st = jnp.dot(st_ref[...], m1_ref[...], preferred_element_type=jnp.float32,
                 precision=jax.lax.Precision.HIGHEST)
    mean = st[0:1, :] / count
    var = st[1:2, :] / count - mean * mean
    a = gb_ref[0:1, :] * jax.lax.rsqrt(var + EPS)
    b = gb_ref[1:2, :] - mean * a
    return a, b


def _accum_stats(st_ref, o):
    d = o.shape[1]
    s1 = jnp.sum(o, axis=0, keepdims=True)
    s2 = jnp.sum(o * o, axis=0, keepdims=True)
    blk = jnp.concatenate([s1, s2, jnp.zeros((6, d), jnp.float32)], 0)

    @pl.when(pl.program_id(0) == 0)
    def _():
        st_ref[...] = jnp.zeros_like(st_ref)

    st_ref[...] += blk


def _rowmask(blk, nreal, mp):
    if mp == nreal:
        return None
    rows = jax.lax.broadcasted_iota(jnp.int32, (blk, 1), 0) \
        + pl.program_id(0) * blk
    return (rows < nreal).astype(jnp.float32)


def _kcon(kpts):
    kc = jnp.zeros((8, 16), jnp.float32)
    kc = kc.at[0:3, 0:15].set(kpts.T)
    kc = kc.at[3, 0:15].set(jnp.sum(kpts * kpts, axis=1))
    kc = kc.at[3, 15].set(1e12)
    kc = kc.at[4, 0:15].set(1.0)
    return kc


def _seg_sum(y3):
    k = y3.shape[1]
    while k > 8:
        k //= 2
        y3 = y3[:, :k, :] + y3[:, k:2 * k, :]
    return jnp.sum(y3, axis=1)


def _edge_wts(pg3, q, kc, sigma, bq, r):
    d3 = pg3[:, :, 0:3] - q[:, None, 0:3]
    d = d3.reshape(r, 3)
    ex = d[:, 0:1] - kc[0:1, :]
    ey = d[:, 1:2] - kc[1:2, :]
    ez = d[:, 2:3] - kc[2:3, :]
    sq = ex * ex + ey * ey + ez * ez
    return jnp.clip(1.0 - jnp.sqrt(sq) / sigma, 0.0, None) * kc[4:5, :]


def _kpconv(z, pg, qp, kpts, sigma, w, mp, nreal, bq, bc, c, d, norm):
    pack = max(1, 128 // c)
    npass = -(-P // pack)
    p_pad = pack * npass
    cpk = c * pack
    wp = jnp.zeros((p_pad, c, d), jnp.float32).at[0:P].set(w).reshape(p_pad * c, d)
    kc_v = _kcon(kpts)
    r = bq * K
    stats, gbv, m1v, count = norm if norm is not None else (None,) * 4
    ej_v = jnp.asarray(
        np.equal(np.arange(cpk)[None, :] // c,
                 np.arange(pack)[:, None]).astype(np.float32))
    tile_v = jnp.asarray(np.tile(np.eye(c, dtype=np.float32), (1, pack)))

    def body(*refs):
        if norm is not None:
            (z_ref, pg_ref, qp_ref, kc_ref, ej_ref, tile_ref, st_in, gb_ref,
             m1_ref, w_ref, o_ref, st_ref) = refs
        else:
            (z_ref, pg_ref, qp_ref, kc_ref, ej_ref, tile_ref, w_ref,
             o_ref, st_ref) = refs
        zv = z_ref[...]
        if norm is not None:
            a, b = _gn_ab(st_in, gb_ref, m1_ref, count)
            zv = _lk(zv * a + b)
        sv = _edge_wts(pg_ref[...].reshape(bq, K, 16), qp_ref[...],
                       kc_ref[...], sigma, bq, r)
        ej = ej_ref[...]
        hs = []
        for t in range(bq // bc):
            rc = bc * K
            z2 = zv[t * rc:(t + 1) * rc, :]
            if pack > 1:
                z2 = jnp.dot(z2, tile_ref[...],
                             preferred_element_type=jnp.float32,
                             precision=jax.lax.Precision.HIGHEST)
            zrep = z2.reshape(bc, K, cpk)
            sc_ = sv[t * rc:(t + 1) * rc, :]
            hts = []
            for j in range(npass):
                srep = jnp.dot(sc_[:, j * pack:(j + 1) * pack], ej,
                               preferred_element_type=jnp.float32,
                               precision=jax.lax.Precision.HIGHEST)
                hts.append(_seg_sum(zrep * srep.reshape(bc, K, cpk)))
            hs.append(jnp.concatenate(hts, axis=1))
        h = jnp.concatenate(hs, axis=0)
        o = jnp.dot(h, w_ref[...], preferred_element_type=jnp.float32,
                 precision=jax.lax.Precision.HIGHEST)
        m = _rowmask(bq, nreal, mp)
        if m is not None:
            o = o * m
        o_ref[...] = o
        _accum_stats(st_ref, o)

    in_specs = [
        pl.BlockSpec((r, c), lambda i: (i, 0)),
        pl.BlockSpec((r, 16), lambda i: (i, 0)),
        pl.BlockSpec((bq, 16), lambda i: (i, 0)),
        pl.BlockSpec((8, 16), lambda i: (0, 0)),
        pl.BlockSpec((pack, cpk), lambda i: (0, 0)),
        pl.BlockSpec((c, cpk), lambda i: (0, 0)),
    ]
    args = [z, pg, qp, kc_v, ej_v, tile_v]
    if norm is not None:
        in_specs += [
            pl.BlockSpec((8, c), lambda i: (0, 0)),
            pl.BlockSpec((8, c), lambda i: (0, 0)),
            pl.BlockSpec((c, c), lambda i: (0, 0)),
        ]
        args += [stats, gbv, m1v]
    in_specs.append(pl.BlockSpec((p_pad * c, d), lambda i: (0, 0)))
    args.append(wp)
    return pl.pallas_call(
        body,
        grid=(mp // bq,),
        in_specs=in_specs,
        out_specs=[
            pl.BlockSpec((bq, d), lambda i: (i, 0)),
            pl.BlockSpec((8, d), lambda i: (0, 0)),
        ],
        out_shape=[
            jax.ShapeDtypeStruct((mp, d), jnp.float32),
            jax.ShapeDtypeStruct((8, d), jnp.float32),
        ],
    )(*args)


def _kpconv_c0(pg, qp, kpts, sigma, w, m, bq):
    d = w.shape[2]
    w16 = jnp.zeros((16, d), jnp.float32).at[0:P].set(w[:, 0, :])
    kc_v = _kcon(kpts)
    r = bq * K

    def body(pg_ref, qp_ref, kc_ref, w_ref, o_ref, st_ref):
        pg2 = pg_ref[...]
        pg3 = pg2.reshape(bq, K, 16)
        sv = _edge_wts(pg3, qp_ref[...], kc_ref[...], sigma, bq, r)
        y2 = pg2[:, 3:4] * sv
        h = _seg_sum(y2.reshape(bq, K, 16))
        o = jnp.dot(h, w_ref[...], preferred_element_type=jnp.float32,
                 precision=jax.lax.Precision.HIGHEST)
        o_ref[...] = o
        _accum_stats(st_ref, o)

    return pl.pallas_call(
        body,
        grid=(m // bq,),
        in_specs=[
            pl.BlockSpec((r, 16), lambda i: (i, 0)),
            pl.BlockSpec((bq, 16), lambda i: (i, 0)),
            pl.BlockSpec((8, 16), lambda i: (0, 0)),
            pl.BlockSpec((16, d), lambda i: (0, 0)),
        ],
        out_specs=[
            pl.BlockSpec((bq, d), lambda i: (i, 0)),
            pl.BlockSpec((8, d), lambda i: (0, 0)),
        ],
        out_shape=[
            jax.ShapeDtypeStruct((m, d), jnp.float32),
            jax.ShapeDtypeStruct((8, d), jnp.float32),
        ],
    )(pg, qp, kc_v, w16)


def _dense(xs, ws, b, mp, nreal, blk, norm=None, want_stats=True):
    d = ws[0].shape[1]
    b2 = b.reshape(1, d)
    nx = len(xs)
    stats, gbv, m1v, count = norm if norm is not None else (None,) * 4
    cn = xs[0].shape[1]

    def body(*refs):
        i = 0
        x_refs = refs[i:i + nx]; i += nx
        if norm is not None:
            st_in, gb_ref, m1_ref = refs[i:i + 3]; i += 3
        w_refs = refs[i:i + nx]; i += nx
        b_ref = refs[i]; i += 1
        if want_stats:
            o_ref, st_ref = refs[i:i + 2]
        else:
            o_ref = refs[i]
        x0 = x_refs[0][...]
        if norm is not None:
            a, bb = _gn_ab(st_in, gb_ref, m1_ref, count)
            x0 = _lk(x0 * a + bb)
        o = jnp.dot(x0, w_refs[0][...], preferred_element_type=jnp.float32,
                 precision=jax.lax.Precision.HIGHEST)
        for j in range(1, nx):
            o += jnp.dot(x_refs[j][...], w_refs[j][...],
                         preferred_element_type=jnp.float32,
                 precision=jax.lax.Precision.HIGHEST)
        o += b_ref[...]
        m = _rowmask(blk, nreal, mp)
        if m is not None:
            o = o * m
        o_ref[...] = o
        if want_stats:
            _accum_stats(st_ref, o)

    in_specs = [pl.BlockSpec((blk, x.shape[1]), lambda i: (i, 0)) for x in xs]
    args = list(xs)
    if norm is not None:
        in_specs += [
            pl.BlockSpec((8, cn), lambda i: (0, 0)),
            pl.BlockSpec((8, cn), lambda i: (0, 0)),
            pl.BlockSpec((cn, cn), lambda i: (0, 0)),
        ]
        args += [stats, gbv, m1v]
    in_specs += [pl.BlockSpec((w.shape[0], d), lambda i: (0, 0)) for w in ws]
    args += list(ws)
    in_specs.append(pl.BlockSpec((1, d), lambda i: (0, 0)))
    args.append(b2)
    if want_stats:
        out_specs = [
            pl.BlockSpec((blk, d), lambda i: (i, 0)),
            pl.BlockSpec((8, d), lambda i: (0, 0)),
        ]
        out_shape = [
            jax.ShapeDtypeStruct((mp, d), jnp.float32),
            jax.ShapeDtypeStruct((8, d), jnp.float32),
        ]
    else:
        out_specs = pl.BlockSpec((blk, d), lambda i: (i, 0))
        out_shape = jax.ShapeDtypeStruct((mp, d), jnp.float32)
    return pl.pallas_call(
        body,
        grid=(mp // blk,),
        in_specs=in_specs,
        out_specs=out_specs,
        out_shape=out_shape,
    )(*args)


def _finalize(raw, stats, gbv, mp, nreal, blk, count):
    d = raw.shape[1]
    m1v = _m1(d)

    def body(x_ref, st_ref, gb_ref, m1_ref, o_ref):
        a, b = _gn_ab(st_ref, gb_ref, m1_ref, count)
        o = _lk(x_ref[...] * a + b)
        m = _rowmask(blk, nreal, mp)
        if m is not None:
            o = o * m
        o_ref[...] = o

    return pl.pallas_call(
        body,
        grid=(mp // blk,),
        in_specs=[
            pl.BlockSpec((blk, d), lambda i: (i, 0)),
            pl.BlockSpec((8, d), lambda i: (0, 0)),
            pl.BlockSpec((8, d), lambda i: (0, 0)),
            pl.BlockSpec((d, d), lambda i: (0, 0)),
        ],
        out_specs=pl.BlockSpec((blk, d), lambda i: (i, 0)),
        out_shape=jax.ShapeDtypeStruct((mp, d), jnp.float32),
    )(raw, stats, gbv, m1v)


def _combine(fraw, fstats, fgb, sc, scnorm, mp, nreal, blk, count):
    d = fraw.shape[1]
    m1v = _m1(d)

    def body(*refs):
        if scnorm is not None:
            f_ref, fst, fgb_ref, s_ref, sst, sgb_ref, m1_ref, o_ref = refs
        else:
            f_ref, fst, fgb_ref, s_ref, m1_ref, o_ref = refs
        a1, b1 = _gn_ab(fst, fgb_ref, m1_ref, count)
        f = f_ref[...] * a1 + b1
        if scnorm is not None:
            a2, b2 = _gn_ab(sst, sgb_ref, m1_ref, count)
            s = s_ref[...] * a2 + b2
        else:
            s = s_ref[...]
        o = _lk(f + s)
        m = _rowmask(blk, nreal, mp)
        if m is not None:
            o = o * m
        o_ref[...] = o

    in_specs = [
        pl.BlockSpec((blk, d), lambda i: (i, 0)),
        pl.BlockSpec((8, d), lambda i: (0, 0)),
        pl.BlockSpec((8, d), lambda i: (0, 0)),
        pl.BlockSpec((blk, d), lambda i: (i, 0)),
    ]
    args = [fraw, fstats, fgb, sc]
    if scnorm is not None:
        in_specs += [
            pl.BlockSpec((8, d), lambda i: (0, 0)),
            pl.BlockSpec((8, d), lambda i: (0, 0)),
        ]
        args += list(scnorm)
    in_specs.append(pl.BlockSpec((d, d), lambda i: (0, 0)))
    args.append(m1v)
    return pl.pallas_call(
        body,
        grid=(mp // blk,),
        in_specs=in_specs,
        out_specs=pl.BlockSpec((blk, d), lambda i: (i, 0)),
        out_shape=jax.ShapeDtypeStruct((mp, d), jnp.float32),
    )(*args)


def _maxpool(zg, mp, nreal, bq, c):
    r = bq * K

    def body(z_ref, o_ref):
        z3 = z_ref[...].reshape(bq, K, c)
        o = jnp.max(z3, axis=1)
        m = _rowmask(bq, nreal, mp)
        if m is not None:
            o = o * m
        o_ref[...] = o

    return pl.pallas_call(
        body,
        grid=(mp // bq,),
        in_specs=[pl.BlockSpec((r, c), lambda i: (i, 0))],
        out_specs=pl.BlockSpec((bq, c), lambda i: (i, 0)),
        out_shape=jax.ShapeDtypeStruct((mp, c), jnp.float32),
    )(zg)


_LV = {
    0: dict(n=10000, mp=10000, blk=1000, bq=200, bc=200),
    1: dict(n=2500, mp=2560, blk=512, bq=128, bc=128),
    2: dict(n=625, mp=640, blk=640, bq=128, bc=128),
}


def _res_block(x, pg, qp, idx, p, sigma, lq, ls, strided):
    q = _LV[lq]
    s = _LV[ls]
    cin = x.shape[1]
    mid = p['u1']['lin']['w'].shape[0]
    cout = p['u2']['lin']['w'].shape[0]
    u1raw, u1st = _dense([x], [p['u1']['lin']['w'].T], p['u1']['lin']['b'],
                         s['mp'], s['n'], s['blk'])
    z = _sc_gather(u1raw, idx)
    kpraw, kpst = _kpconv(z, pg, qp, p['kp']['kpts'], sigma, p['kp']['w'],
                          q['mp'], q['n'], q['bq'], q['bc'], mid, mid,
                          (u1st, _gb(p['u1']['gn']), _m1(mid),
                           float(s['n'] * (mid // G))))
    u2raw, u2st = _dense([kpraw], [p['u2']['lin']['w'].T], p['u2']['lin']['b'],
                         q['mp'], q['n'], q['blk'],
                         norm=(kpst, _gb(p['gnc']), _m1(mid),
                               float(q['n'] * (mid // G))))
    count = float(q['n'] * (cout // G))
    if strided:
        zm = _sc_gather(x, idx)
        scf = _maxpool(zm, q['mp'], q['n'], q['bq'], cin)
        return _combine(u2raw, u2st, _gb(p['u2']['gn']), scf, None,
                        q['mp'], q['n'], q['blk'], count)
    scraw, scst = _dense([x], [p['usc']['lin']['w'].T], p['usc']['lin']['b'],
                         q['mp'], q['n'], q['blk'])
    return _combine(u2raw, u2st, _gb(p['u2']['gn']), scraw,
                    (scst, _gb(p['usc']['gn'])), q['mp'], q['n'], q['blk'],
                    count)


def _pad_rows(a, mp):
    n = a.shape[0]
    if mp == n:
        return a
    return jnp.concatenate(
        [a, jnp.zeros((mp - n,) + a.shape[1:], a.dtype)], axis=0)


def kernel(feats, points_0, points_1, points_2, neighbors_0, neighbors_1,
           neighbors_2, subsampling_0, subsampling_1, upsampling_0,
           upsampling_1, params):
    n0 = points_0.shape[0]

    t0 = jnp.concatenate([points_0, feats,
                          jnp.zeros((n0, 12), jnp.float32)], axis=1)
    t1 = _pad_rows(jnp.concatenate(
        [points_1, jnp.zeros((_LV[1]['n'], 13), jnp.float32)], axis=1),
        _LV[1]['mp'])
    t2 = _pad_rows(jnp.concatenate(
        [points_2, jnp.zeros((_LV[2]['n'], 13), jnp.float32)], axis=1),
        _LV[2]['mp'])

    i_n0 = neighbors_0.reshape(-1)
    i_s0 = _pad_rows(subsampling_0.reshape(-1), _LV[1]['mp'] * K)
    i_n1 = _pad_rows(neighbors_1.reshape(-1), _LV[1]['mp'] * K)
    i_s1 = _pad_rows(subsampling_1.reshape(-1), _LV[2]['mp'] * K)
    i_n2 = _pad_rows(neighbors_2.reshape(-1), _LV[2]['mp'] * K)
    i_u1 = _pad_rows(upsampling_1[:, 0], _LV[1]['mp'])
    i_u0 = upsampling_0[:, 0]

    pg0 = _sc_gather(t0, i_n0)
    pgs0 = _sc_gather(t0, i_s0)
    pg1 = _sc_gather(t1, i_n1)
    pgs1 = _sc_gather(t1, i_s1)
    pg2 = _sc_gather(t2, i_n2)

    pr = params
    c0raw, c0st = _kpconv_c0(pg0, t0, pr['c0']['kp']['kpts'], 0.1,
                             pr['c0']['kp']['w'], n0, 200)
    x = _finalize(c0raw, c0st, _gb(pr['c0']['gn']), n0, n0, 1000,
                  float(n0 * (64 // G)))

    x = _res_block(x, pg0, t0, i_n0, pr['r01'], 0.1, 0, 0, False)
    e0 = x
    x = _res_block(x, pgs0, t1, i_s0, pr['r10'], 0.1, 1, 0, True)
    x = _res_block(x, pg1, t1, i_n1, pr['r11'], 0.2, 1, 1, False)
    e1 = x
    x = _res_block(x, pgs1, t2, i_s1, pr['r20'], 0.2, 2, 1, True)
    x = _res_block(x, pg2, t2, i_n2, pr['r21'], 0.4, 2, 2, False)
    e2 = x

    u1g = _sc_gather(e2, i_u1)
    d0w = pr['d0']['lin']['w']
    d0raw, d0st = _dense([u1g, e1], [d0w[:, :512].T, d0w[:, 512:].T],
                         pr['d0']['lin']['b'], _LV[1]['mp'], _LV[1]['n'],
                         _LV[1]['blk'])
    d_mid = _finalize(d0raw, d0st, _gb(pr['d0']['gn']), _LV[1]['mp'],
                      _LV[1]['n'], _LV[1]['blk'],
                      float(_LV[1]['n'] * (256 // G)))

    u0g = _sc_gather(d_mid, i_u0)
    d1w = pr['d1']['lin']['w']
    d_fine = _dense([u0g, e0], [d1w[:, :256].T, d1w[:, 256:].T],
                    pr['d1']['lin']['b'], n0, n0, 1000, want_stats=False)

    return (d_fine, d_mid[:_LV[1]['n']], e2[:_LV[2]['n']])


# speedup vs baseline: 1.4832x; 1.4791x over previous
import functools

import jax
import jax.numpy as jnp
import numpy as np
from jax.experimental import pallas as pl
from jax.experimental.pallas import tpu as pltpu
from jax.experimental.pallas import tpu_sc as plsc

P = 15
G = 32
SLOPE = 0.1
EPS = 1e-5
K = 32


def _lk(x):
    return jnp.where(x >= 0, x, SLOPE * x)


def _ceil_to(n, m):
    return ((n + m - 1) // m) * m


def _gather_window(e, d):
    cap = 128 if d <= 128 else (64 if d <= 256 else 32)
    for w in range(cap, 0, -8):
        if e % w == 0:
            return w
    return cap


def _sc_gather(table, idx):
    e = idx.shape[0]
    d = table.shape[1]
    mesh = plsc.VectorSubcoreMesh(core_axis_name="core",
                                  subcore_axis_name="subcore")
    cp = pltpu.CompilerParams(use_tc_tiling_on_sc=False)

    if d <= 32:
        ep = _ceil_to(e, 1024)
        if ep != e:
            idx = jnp.concatenate([idx, jnp.zeros((ep - e,), jnp.int32)])
        idx2 = idx.reshape(ep // 128, 128)

        @functools.partial(
            pl.kernel,
            out_type=jax.ShapeDtypeStruct((ep, d), table.dtype),
            mesh=mesh,
            compiler_params=cp,
        )
        def gk8(x_hbm, i_hbm, o_hbm):
            def body(i_vmem, o_vmem):
                for t in range(8):
                    pltpu.sync_copy(x_hbm.at[i_vmem.at[t]],
                                    o_vmem.at[pl.ds(t * 128, 128)])

            pltpu.emit_pipeline(
                body,
                grid=(ep // 1024,),
                in_specs=[pl.BlockSpec((8, 128), lambda i: (i, 0))],
                out_specs=[pl.BlockSpec((1024, d), lambda i: (i, 0))],
                core_axis_name=("core", "subcore"),
                dimension_semantics=(pltpu.PARALLEL,),
            )(i_hbm, o_hbm)

        return gk8(table, idx2)

    window = _gather_window(e, d)
    ep = _ceil_to(e, window)
    if ep != e:
        idx = jnp.concatenate([idx, jnp.zeros((ep - e,), jnp.int32)])
    idx2 = idx.reshape(1, ep)

    @functools.partial(
        pl.kernel,
        out_type=jax.ShapeDtypeStruct((ep, d), table.dtype),
        mesh=mesh,
        compiler_params=cp,
    )
    def gk(x_hbm, i_hbm, o_hbm):
        def body(i_vmem, o_vmem):
            pltpu.sync_copy(x_hbm.at[i_vmem.at[0]], o_vmem)

        pltpu.emit_pipeline(
            body,
            grid=(ep // window,),
            in_specs=[pl.BlockSpec((1, window), lambda i: (0, i))],
            out_specs=[pl.BlockSpec((window, d), lambda i: (i, 0))],
            core_axis_name=("core", "subcore"),
            dimension_semantics=(pltpu.PARALLEL,),
        )(i_hbm, o_hbm)

    out = gk(table, idx2)
    if ep != e:
        out = out[:e]
    return out


def _m1(c):
    gs = c // G
    return jnp.asarray(np.kron(np.eye(G, dtype=np.float32),
                               np.ones((gs, gs), np.float32)))


def _gb(gn):
    c = gn['g'].shape[0]
    z = jnp.zeros((6, c), jnp.float32)
    return jnp.concatenate([gn['g'].reshape(1, c), gn['be'].reshape(1, c), z], 0)


def _gn_ab(st_ref, gb_ref, m1_ref, count):
    st = jnp.dot(st_ref[...], m1_ref[...], preferred_element_type=jnp.float32,
                 precision=jax.lax.Precision.HIGHEST)
    mean = st[0:1, :] / count
    var = st[1:2, :] / count - mean * mean
    a = gb_ref[0:1, :] * jax.lax.rsqrt(var + EPS)
    b = gb_ref[1:2, :] - mean * a
    return a, b


def _accum_stats(st_ref, o):
    d = o.shape[1]
    s1 = jnp.sum(o, axis=0, keepdims=True)
    s2 = jnp.sum(o * o, axis=0, keepdims=True)
    blk = jnp.concatenate([s1, s2, jnp.zeros((6, d), jnp.float32)], 0)

    @pl.when(pl.program_id(0) == 0)
    def _():
        st_ref[...] = jnp.zeros_like(st_ref)

    st_ref[...] += blk


def _rowmask(blk, nreal, mp):
    if mp == nreal:
        return None
    rows = jax.lax.broadcasted_iota(jnp.int32, (blk, 1), 0) \
        + pl.program_id(0) * blk
    return (rows < nreal).astype(jnp.float32)


def _kcon(kpts):
    kc = jnp.zeros((8, 16), jnp.float32)
    kc = kc.at[0:3, 0:15].set(kpts.T)
    kc = kc.at[3, 0:15].set(jnp.sum(kpts * kpts, axis=1))
    kc = kc.at[3, 15].set(1e12)
    kc = kc.at[4, 0:15].set(1.0)
    return kc


def _xdot(a, b01):
    hi = a.astype(jnp.bfloat16)
    lo = (a - hi.astype(jnp.float32)).astype(jnp.bfloat16)
    bb = b01.astype(jnp.bfloat16)
    kw = dict(preferred_element_type=jnp.float32,
              precision=jax.lax.Precision.DEFAULT)
    return jnp.dot(hi, bb, **kw) + jnp.dot(lo, bb, **kw)


def _seg_sum(y3):
    k = y3.shape[1]
    while k > 8:
        k //= 2
        y3 = y3[:, :k, :] + y3[:, k:2 * k, :]
    return jnp.sum(y3, axis=1)


def _edge_wts(pg3, q, kc, sigma, bq, r):
    d3 = pg3[:, :, 0:3] - q[:, None, 0:3]
    d = d3.reshape(r, 3)
    ex = d[:, 0:1] - kc[0:1, :]
    ey = d[:, 1:2] - kc[1:2, :]
    ez = d[:, 2:3] - kc[2:3, :]
    sq = ex * ex + ey * ey + ez * ez
    return jnp.clip(1.0 - jnp.sqrt(sq) / sigma, 0.0, None) * kc[4:5, :]


def _kpconv(z, pg, qp, kpts, sigma, w, mp, nreal, bq, bc, c, d, norm):
    pack = max(1, 128 // c)
    npass = -(-P // pack)
    p_pad = pack * npass
    cpk = c * pack
    wp = jnp.zeros((p_pad, c, d), jnp.float32).at[0:P].set(w).reshape(p_pad * c, d)
    kc_v = _kcon(kpts)
    r = bq * K
    stats, gbv, m1v, count = norm if norm is not None else (None,) * 4
    ej_v = jnp.asarray(
        np.equal(np.arange(cpk)[None, :] // c,
                 np.arange(pack)[:, None]).astype(np.float32))
    tile_v = jnp.asarray(np.tile(np.eye(c, dtype=np.float32), (1, pack)))

    def body(*refs):
        if norm is not None:
            (z_ref, pg_ref, qp_ref, kc_ref, ej_ref, tile_ref, st_in, gb_ref,
             m1_ref, w_ref, o_ref, st_ref) = refs
        else:
            (z_ref, pg_ref, qp_ref, kc_ref, ej_ref, tile_ref, w_ref,
             o_ref, st_ref) = refs
        zv = z_ref[...]
        if norm is not None:
            a, b = _gn_ab(st_in, gb_ref, m1_ref, count)
            zv = _lk(zv * a + b)
        sv = _edge_wts(pg_ref[...].reshape(bq, K, 16), qp_ref[...],
                       kc_ref[...], sigma, bq, r)
        ej = ej_ref[...]
        hs = []
        for t in range(bq // bc):
            rc = bc * K
            z2 = zv[t * rc:(t + 1) * rc, :]
            if pack > 1:
                z2 = _xdot(z2, tile_ref[...])
            zrep = z2.reshape(bc, K, cpk)
            sc_ = sv[t * rc:(t + 1) * rc, :]
            hts = []
            for j in range(npass):
                srep = _xdot(sc_[:, j * pack:(j + 1) * pack], ej)
                hts.append(_seg_sum(zrep * srep.reshape(bc, K, cpk)))
            hs.append(jnp.concatenate(hts, axis=1))
        h = jnp.concatenate(hs, axis=0)
        o = jnp.dot(h, w_ref[...], preferred_element_type=jnp.float32,
                 precision=jax.lax.Precision.HIGHEST)
        m = _rowmask(bq, nreal, mp)
        if m is not None:
            o = o * m
        o_ref[...] = o
        _accum_stats(st_ref, o)

    in_specs = [
        pl.BlockSpec((r, c), lambda i: (i, 0)),
        pl.BlockSpec((r, 16), lambda i: (i, 0)),
        pl.BlockSpec((bq, 16), lambda i: (i, 0)),
        pl.BlockSpec((8, 16), lambda i: (0, 0)),
        pl.BlockSpec((pack, cpk), lambda i: (0, 0)),
        pl.BlockSpec((c, cpk), lambda i: (0, 0)),
    ]
    args = [z, pg, qp, kc_v, ej_v, tile_v]
    if norm is not None:
        in_specs += [
            pl.BlockSpec((8, c), lambda i: (0, 0)),
            pl.BlockSpec((8, c), lambda i: (0, 0)),
            pl.BlockSpec((c, c), lambda i: (0, 0)),
        ]
        args += [stats, gbv, m1v]
    in_specs.append(pl.BlockSpec((p_pad * c, d), lambda i: (0, 0)))
    args.append(wp)
    return pl.pallas_call(
        body,
        grid=(mp // bq,),
        in_specs=in_specs,
        out_specs=[
            pl.BlockSpec((bq, d), lambda i: (i, 0)),
            pl.BlockSpec((8, d), lambda i: (0, 0)),
        ],
        out_shape=[
            jax.ShapeDtypeStruct((mp, d), jnp.float32),
            jax.ShapeDtypeStruct((8, d), jnp.float32),
        ],
    )(*args)


def _kpconv_c0(pg, qp, kpts, sigma, w, m, bq):
    d = w.shape[2]
    w16 = jnp.zeros((16, d), jnp.float32).at[0:P].set(w[:, 0, :])
    kc_v = _kcon(kpts)
    r = bq * K

    def body(pg_ref, qp_ref, kc_ref, w_ref, o_ref, st_ref):
        pg2 = pg_ref[...]
        pg3 = pg2.reshape(bq, K, 16)
        sv = _edge_wts(pg3, qp_ref[...], kc_ref[...], sigma, bq, r)
        y2 = pg2[:, 3:4] * sv
        h = _seg_sum(y2.reshape(bq, K, 16))
        o = jnp.dot(h, w_ref[...], preferred_element_type=jnp.float32,
                 precision=jax.lax.Precision.HIGHEST)
        o_ref[...] = o
        _accum_stats(st_ref, o)

    return pl.pallas_call(
        body,
        grid=(m // bq,),
        in_specs=[
            pl.BlockSpec((r, 16), lambda i: (i, 0)),
            pl.BlockSpec((bq, 16), lambda i: (i, 0)),
            pl.BlockSpec((8, 16), lambda i: (0, 0)),
            pl.BlockSpec((16, d), lambda i: (0, 0)),
        ],
        out_specs=[
            pl.BlockSpec((bq, d), lambda i: (i, 0)),
            pl.BlockSpec((8, d), lambda i: (0, 0)),
        ],
        out_shape=[
            jax.ShapeDtypeStruct((m, d), jnp.float32),
            jax.ShapeDtypeStruct((8, d), jnp.float32),
        ],
    )(pg, qp, kc_v, w16)


def _dense(xs, ws, b, mp, nreal, blk, norm=None, want_stats=True):
    d = ws[0].shape[1]
    b2 = b.reshape(1, d)
    nx = len(xs)
    stats, gbv, m1v, count = norm if norm is not None else (None,) * 4
    cn = xs[0].shape[1]

    def body(*refs):
        i = 0
        x_refs = refs[i:i + nx]; i += nx
        if norm is not None:
            st_in, gb_ref, m1_ref = refs[i:i + 3]; i += 3
        w_refs = refs[i:i + nx]; i += nx
        b_ref = refs[i]; i += 1
        if want_stats:
            o_ref, st_ref = refs[i:i + 2]
        else:
            o_ref = refs[i]
        x0 = x_refs[0][...]
        if norm is not None:
            a, bb = _gn_ab(st_in, gb_ref, m1_ref, count)
            x0 = _lk(x0 * a + bb)
        o = jnp.dot(x0, w_refs[0][...], preferred_element_type=jnp.float32,
                 precision=jax.lax.Precision.HIGHEST)
        for j in range(1, nx):
            o += jnp.dot(x_refs[j][...], w_refs[j][...],
                         preferred_element_type=jnp.float32,
                 precision=jax.lax.Precision.HIGHEST)
        o += b_ref[...]
        m = _rowmask(blk, nreal, mp)
        if m is not None:
            o = o * m
        o_ref[...] = o
        if want_stats:
            _accum_stats(st_ref, o)

    in_specs = [pl.BlockSpec((blk, x.shape[1]), lambda i: (i, 0)) for x in xs]
    args = list(xs)
    if norm is not None:
        in_specs += [
            pl.BlockSpec((8, cn), lambda i: (0, 0)),
            pl.BlockSpec((8, cn), lambda i: (0, 0)),
            pl.BlockSpec((cn, cn), lambda i: (0, 0)),
        ]
        args += [stats, gbv, m1v]
    in_specs += [pl.BlockSpec((w.shape[0], d), lambda i: (0, 0)) for w in ws]
    args += list(ws)
    in_specs.append(pl.BlockSpec((1, d), lambda i: (0, 0)))
    args.append(b2)
    if want_stats:
        out_specs = [
            pl.BlockSpec((blk, d), lambda i: (i, 0)),
            pl.BlockSpec((8, d), lambda i: (0, 0)),
        ]
        out_shape = [
            jax.ShapeDtypeStruct((mp, d), jnp.float32),
            jax.ShapeDtypeStruct((8, d), jnp.float32),
        ]
    else:
        out_specs = pl.BlockSpec((blk, d), lambda i: (i, 0))
        out_shape = jax.ShapeDtypeStruct((mp, d), jnp.float32)
    return pl.pallas_call(
        body,
        grid=(mp // blk,),
        in_specs=in_specs,
        out_specs=out_specs,
        out_shape=out_shape,
    )(*args)


def _finalize(raw, stats, gbv, mp, nreal, blk, count):
    d = raw.shape[1]
    m1v = _m1(d)

    def body(x_ref, st_ref, gb_ref, m1_ref, o_ref):
        a, b = _gn_ab(st_ref, gb_ref, m1_ref, count)
        o = _lk(x_ref[...] * a + b)
        m = _rowmask(blk, nreal, mp)
        if m is not None:
            o = o * m
        o_ref[...] = o

    return pl.pallas_call(
        body,
        grid=(mp // blk,),
        in_specs=[
            pl.BlockSpec((blk, d), lambda i: (i, 0)),
            pl.BlockSpec((8, d), lambda i: (0, 0)),
            pl.BlockSpec((8, d), lambda i: (0, 0)),
            pl.BlockSpec((d, d), lambda i: (0, 0)),
        ],
        out_specs=pl.BlockSpec((blk, d), lambda i: (i, 0)),
        out_shape=jax.ShapeDtypeStruct((mp, d), jnp.float32),
    )(raw, stats, gbv, m1v)


def _combine(fraw, fstats, fgb, sc, scnorm, mp, nreal, blk, count):
    d = fraw.shape[1]
    m1v = _m1(d)

    def body(*refs):
        if scnorm is not None:
            f_ref, fst, fgb_ref, s_ref, sst, sgb_ref, m1_ref, o_ref = refs
        else:
            f_ref, fst, fgb_ref, s_ref, m1_ref, o_ref = refs
        a1, b1 = _gn_ab(fst, fgb_ref, m1_ref, count)
        f = f_ref[...] * a1 + b1
        if scnorm is not None:
            a2, b2 = _gn_ab(sst, sgb_ref, m1_ref, count)
            s = s_ref[...] * a2 + b2
        else:
            s = s_ref[...]
        o = _lk(f + s)
        m = _rowmask(blk, nreal, mp)
        if m is not None:
            o = o * m
        o_ref[...] = o

    in_specs = [
        pl.BlockSpec((blk, d), lambda i: (i, 0)),
        pl.BlockSpec((8, d), lambda i: (0, 0)),
        pl.BlockSpec((8, d), lambda i: (0, 0)),
        pl.BlockSpec((blk, d), lambda i: (i, 0)),
    ]
    args = [fraw, fstats, fgb, sc]
    if scnorm is not None:
        in_specs += [
            pl.BlockSpec((8, d), lambda i: (0, 0)),
            pl.BlockSpec((8, d), lambda i: (0, 0)),
        ]
        args += list(scnorm)
    in_specs.append(pl.BlockSpec((d, d), lambda i: (0, 0)))
    args.append(m1v)
    return pl.pallas_call(
        body,
        grid=(mp // blk,),
        in_specs=in_specs,
        out_specs=pl.BlockSpec((blk, d), lambda i: (i, 0)),
        out_shape=jax.ShapeDtypeStruct((mp, d), jnp.float32),
    )(*args)


def _maxpool(zg, mp, nreal, bq, c):
    r = bq * K

    def body(z_ref, o_ref):
        z3 = z_ref[...].reshape(bq, K, c)
        o = jnp.max(z3, axis=1)
        m = _rowmask(bq, nreal, mp)
        if m is not None:
            o = o * m
        o_ref[...] = o

    return pl.pallas_call(
        body,
        grid=(mp // bq,),
        in_specs=[pl.BlockSpec((r, c), lambda i: (i, 0))],
        out_specs=pl.BlockSpec((bq, c), lambda i: (i, 0)),
        out_shape=jax.ShapeDtypeStruct((mp, c), jnp.float32),
    )(zg)


_LV = {
    0: dict(n=10000, mp=10000, blk=1000, bq=200, bc=200),
    1: dict(n=2500, mp=2560, blk=512, bq=128, bc=128),
    2: dict(n=625, mp=640, blk=640, bq=128, bc=128),
}


def _res_block(x, pg, qp, idx, p, sigma, lq, ls, strided):
    q = _LV[lq]
    s = _LV[ls]
    cin = x.shape[1]
    mid = p['u1']['lin']['w'].shape[0]
    cout = p['u2']['lin']['w'].shape[0]
    u1raw, u1st = _dense([x], [p['u1']['lin']['w'].T], p['u1']['lin']['b'],
                         s['mp'], s['n'], s['blk'])
    z = _sc_gather(u1raw, idx)
    kpraw, kpst = _kpconv(z, pg, qp, p['kp']['kpts'], sigma, p['kp']['w'],
                          q['mp'], q['n'], q['bq'], q['bc'], mid, mid,
                          (u1st, _gb(p['u1']['gn']), _m1(mid),
                           float(s['n'] * (mid // G))))
    u2raw, u2st = _dense([kpraw], [p['u2']['lin']['w'].T], p['u2']['lin']['b'],
                         q['mp'], q['n'], q['blk'],
                         norm=(kpst, _gb(p['gnc']), _m1(mid),
                               float(q['n'] * (mid // G))))
    count = float(q['n'] * (cout // G))
    if strided:
        zm = _sc_gather(x, idx)
        scf = _maxpool(zm, q['mp'], q['n'], q['bq'], cin)
        return _combine(u2raw, u2st, _gb(p['u2']['gn']), scf, None,
                        q['mp'], q['n'], q['blk'], count)
    scraw, scst = _dense([x], [p['usc']['lin']['w'].T], p['usc']['lin']['b'],
                         q['mp'], q['n'], q['blk'])
    return _combine(u2raw, u2st, _gb(p['u2']['gn']), scraw,
                    (scst, _gb(p['usc']['gn'])), q['mp'], q['n'], q['blk'],
                    count)


def _pad_rows(a, mp):
    n = a.shape[0]
    if mp == n:
        return a
    return jnp.concatenate(
        [a, jnp.zeros((mp - n,) + a.shape[1:], a.dtype)], axis=0)


def kernel(feats, points_0, points_1, points_2, neighbors_0, neighbors_1,
           neighbors_2, subsampling_0, subsampling_1, upsampling_0,
           upsampling_1, params):
    n0 = points_0.shape[0]

    t0 = jnp.concatenate([points_0, feats,
                          jnp.zeros((n0, 12), jnp.float32)], axis=1)
    t1 = _pad_rows(jnp.concatenate(
        [points_1, jnp.zeros((_LV[1]['n'], 13), jnp.float32)], axis=1),
        _LV[1]['mp'])
    t2 = _pad_rows(jnp.concatenate(
        [points_2, jnp.zeros((_LV[2]['n'], 13), jnp.float32)], axis=1),
        _LV[2]['mp'])

    i_n0 = neighbors_0.reshape(-1)
    i_s0 = _pad_rows(subsampling_0.reshape(-1), _LV[1]['mp'] * K)
    i_n1 = _pad_rows(neighbors_1.reshape(-1), _LV[1]['mp'] * K)
    i_s1 = _pad_rows(subsampling_1.reshape(-1), _LV[2]['mp'] * K)
    i_n2 = _pad_rows(neighbors_2.reshape(-1), _LV[2]['mp'] * K)
    i_u1 = _pad_rows(upsampling_1[:, 0], _LV[1]['mp'])
    i_u0 = upsampling_0[:, 0]

    pg0 = _sc_gather(t0, i_n0)
    pgs0 = _sc_gather(t0, i_s0)
    pg1 = _sc_gather(t1, i_n1)
    pgs1 = _sc_gather(t1, i_s1)
    pg2 = _sc_gather(t2, i_n2)

    pr = params
    c0raw, c0st = _kpconv_c0(pg0, t0, pr['c0']['kp']['kpts'], 0.1,
                             pr['c0']['kp']['w'], n0, 200)
    x = _finalize(c0raw, c0st, _gb(pr['c0']['gn']), n0, n0, 1000,
                  float(n0 * (64 // G)))

    x = _res_block(x, pg0, t0, i_n0, pr['r01'], 0.1, 0, 0, False)
    e0 = x
    x = _res_block(x, pgs0, t1, i_s0, pr['r10'], 0.1, 1, 0, True)
    x = _res_block(x, pg1, t1, i_n1, pr['r11'], 0.2, 1, 1, False)
    e1 = x
    x = _res_block(x, pgs1, t2, i_s1, pr['r20'], 0.2, 2, 1, True)
    x = _res_block(x, pg2, t2, i_n2, pr['r21'], 0.4, 2, 2, False)
    e2 = x

    u1g = _sc_gather(e2, i_u1)
    d0w = pr['d0']['lin']['w']
    d0raw, d0st = _dense([u1g, e1], [d0w[:, :512].T, d0w[:, 512:].T],
                         pr['d0']['lin']['b'], _LV[1]['mp'], _LV[1]['n'],
                         _LV[1]['blk'])
    d_mid = _finalize(d0raw, d0st, _gb(pr['d0']['gn']), _LV[1]['mp'],
                      _LV[1]['n'], _LV[1]['blk'],
                      float(_LV[1]['n'] * (256 // G)))

    u0g = _sc_gather(d_mid, i_u0)
    d1w = pr['d1']['lin']['w']
    d_fine = _dense([u0g, e0], [d1w[:, :256].T, d1w[:, 256:].T],
                    pr['d1']['lin']['b'], n0, n0, 1000, want_stats=False)

    return (d_fine, d_mid[:_LV[1]['n']], e2[:_LV[2]['n']])
